# baseline (speedup 1.0000x reference)
import jax
import jax.numpy as jnp
from jax import lax
from jax.experimental import pallas as pl
from jax.experimental.pallas import tpu as pltpu
from jax.experimental.pallas import tpu_sc as plsc

B = 16384
L = 100
F = 64
NP = L // 2
NPIECE = 52
NPID = 56
NC, NS = 2, 16
NW = NC * NS
RPW = B // NW
CH = 8
NCHUNK = RPW // CH


def _body(cont_hbm, idx_hbm, ptable_hbm, out_hbm, raw_v, cbuf_v, buf_v, sem_in, sem_out, *pid_refs):
    wid = lax.axis_index("s") * NC + lax.axis_index("c")
    row0 = wid * RPW
    lane = jax.lax.iota(jnp.int32, 16)

    def chunk_body(g, carry):
        base = row0 + g * CH
        blk = base // CH
        pltpu.sync_copy(idx_hbm.at[pl.ds(base, CH)], raw_v)
        for i in range(CH):
            pidr = pid_refs[i]
            rvec = jnp.full((16,), i, dtype=jnp.int32)
            for c0 in (0, 16, 32, 40):
                col = jnp.minimum((c0 + lane) * 2, L - 2)
                t0 = plsc.load_gather(raw_v, [rvec, col])
                t1 = plsc.load_gather(raw_v, [rvec, col + 1])
                pidr[pl.ds(c0, 16)] = t0 * 4 + t1
        descs = [pltpu.async_copy(cont_hbm.at[pl.ds(base, CH)], cbuf_v, sem_in)]
        for i in range(CH):
            descs.append(
                pltpu.async_copy(
                    ptable_hbm.at[pid_refs[i]],
                    buf_v.at[i, pl.ds(2, NPID)],
                    sem_in,
                )
            )
        for d in descs:
            d.wait()
        for i in range(CH):
            for j in range(8):
                buf_v[i, 0, pl.ds(16 * j, 16)] = cbuf_v[i, pl.ds(16 * j, 16)]
                buf_v[i, 1, pl.ds(16 * j, 16)] = cbuf_v[i, pl.ds(128 + 16 * j, 16)]
        out_descs = []
        for i in range(CH):
            out_descs.append(
                pltpu.async_copy(
                    buf_v.at[i, pl.ds(0, NPIECE)], out_hbm.at[blk, :, i], sem_out
                )
            )
        for d in out_descs:
            d.wait()
        return carry

    lax.fori_loop(0, NCHUNK, chunk_body, 0)


def kernel(continuous_fields, tile_type_field, embed_table):
    idx = tile_type_field.astype(jnp.int32)
    d = jnp.arange(16, dtype=jnp.int32)
    ptable = jnp.concatenate(
        [embed_table[jnp.clip(d // 4 + 1, 0, 3)], embed_table[jnp.clip(d % 4 + 1, 0, 3)]],
        axis=1,
    )
    mesh = plsc.VectorSubcoreMesh(core_axis_name="c", subcore_axis_name="s")
    out4 = pl.kernel(
        _body,
        out_type=jax.ShapeDtypeStruct((B // 8, NPIECE, 8, 128), jnp.float32),
        mesh=mesh,
        compiler_params=pltpu.CompilerParams(
            use_tc_tiling_on_sc=False, needs_layout_passes=False
        ),
        scratch_types=[
            pltpu.VMEM((CH, L), jnp.int32),
            pltpu.VMEM((CH, 2 * 128), jnp.float32),
            pltpu.VMEM((CH, 2 + NPID, 128), jnp.float32),
            pltpu.SemaphoreType.DMA,
            pltpu.SemaphoreType.DMA,
        ]
        + [pltpu.VMEM((NPID,), jnp.int32) for _ in range(CH)],
    )(continuous_fields, idx, ptable)
    return out4.transpose(0, 2, 1, 3).reshape(B, NPIECE * 128)


# speedup vs baseline: 3.8777x; 3.8777x over previous
import jax
import jax.numpy as jnp
from jax import lax
from jax.experimental import pallas as pl
from jax.experimental.pallas import tpu as pltpu
from jax.experimental.pallas import tpu_sc as plsc

B = 16384
L = 100
NQ = L // 4
NSLOT = 1 + NQ
NQID = 32
NC, NS = 2, 16
NW = NC * NS
RPW = B // NW
CH = 8
NCHUNK = RPW // CH


def _body(cont_hbm, idx_hbm, qtable_hbm, out_hbm, raw_v, buf_v, sem_in, sem_out, *pid_refs):
    wid = lax.axis_index("s") * NC + lax.axis_index("c")
    row0 = wid * RPW
    lane = jax.lax.iota(jnp.int32, 16)

    def chunk_body(g, carry):
        base = row0 + g * CH
        blk = base // CH
        pltpu.sync_copy(idx_hbm.at[pl.ds(base, CH)], raw_v)
        for i in range(CH):
            pidr = pid_refs[i]
            rvec = jnp.full((16,), i, dtype=jnp.int32)
            for c0 in (0, 16):
                col = jnp.minimum(c0 + lane, NQ - 1) * 4
                t0 = plsc.load_gather(raw_v, [rvec, col])
                t1 = plsc.load_gather(raw_v, [rvec, col + 1])
                t2 = plsc.load_gather(raw_v, [rvec, col + 2])
                t3 = plsc.load_gather(raw_v, [rvec, col + 3])
                pidr[pl.ds(c0, 16)] = ((t0 * 4 + t1) * 4 + t2) * 4 + t3
        descs = []
        for i in range(CH):
            descs.append(
                pltpu.async_copy(cont_hbm.at[base + i], buf_v.at[i, 0], sem_in)
            )
            descs.append(
                pltpu.async_copy(
                    qtable_hbm.at[pid_refs[i]],
                    buf_v.at[i, pl.ds(1, NQID)],
                    sem_in,
                )
            )
        for d in descs:
            d.wait()
        out_descs = []
        for i in range(CH):
            out_descs.append(
                pltpu.async_copy(
                    buf_v.at[i, pl.ds(0, NSLOT)],
                    out_hbm.at[blk, :, :, i],
                    sem_out,
                )
            )
        for d in out_descs:
            d.wait()
        return carry

    lax.fori_loop(0, NCHUNK, chunk_body, 0)


def kernel(continuous_fields, tile_type_field, embed_table):
    idx = tile_type_field.astype(jnp.int32)
    cont = continuous_fields.reshape(B, 2, 128)
    q = jnp.arange(256, dtype=jnp.int32)
    qtable = jnp.concatenate(
        [
            embed_table[jnp.clip((q >> 6) & 3, 0, 2) + 1],
            embed_table[jnp.clip((q >> 4) & 3, 0, 2) + 1],
            embed_table[jnp.clip((q >> 2) & 3, 0, 2) + 1],
            embed_table[jnp.clip(q & 3, 0, 2) + 1],
        ],
        axis=1,
    ).reshape(256, 2, 128)
    mesh = plsc.VectorSubcoreMesh(core_axis_name="c", subcore_axis_name="s")
    out5 = pl.kernel(
        _body,
        out_type=jax.ShapeDtypeStruct((B // 8, NSLOT, 2, 8, 128), jnp.float32),
        mesh=mesh,
        compiler_params=pltpu.CompilerParams(
            use_tc_tiling_on_sc=False, needs_layout_passes=False
        ),
        scratch_types=[
            pltpu.VMEM((CH, L), jnp.int32),
            pltpu.VMEM((CH, 1 + NQID, 2, 128), jnp.float32),
            pltpu.SemaphoreType.DMA,
            pltpu.SemaphoreType.DMA,
        ]
        + [pltpu.VMEM((NQID,), jnp.int32) for _ in range(CH)],
    )(cont, idx, qtable)
    return out5.transpose(0, 3, 1, 2, 4).reshape(B, NSLOT * 256)


# speedup vs baseline: 4.5444x; 1.1719x over previous
import jax
import jax.numpy as jnp
from jax import lax
from jax.experimental import pallas as pl
from jax.experimental.pallas import tpu as pltpu
from jax.experimental.pallas import tpu_sc as plsc

B = 16384
L = 100
NQ = L // 4
NSLOT = 1 + NQ
NQID = 32
NC, NS = 2, 16
NW = NC * NS
RPW = B // NW
CH = 8
NCHUNK = RPW // CH


def _body(
    cont_hbm,
    idx_hbm,
    qtable_hbm,
    out_hbm,
    raw_v,
    buf_v,
    sem_in,
    sem_out0,
    sem_out1,
    *pid_refs,
):
    wid = lax.axis_index("s") * NC + lax.axis_index("c")
    row0 = wid * RPW
    lane = jax.lax.iota(jnp.int32, 16)
    out_sems = (sem_out0, sem_out1)

    def compute_pids(base):
        pltpu.sync_copy(idx_hbm.at[pl.ds(base, CH)], raw_v)
        for i in range(CH):
            pidr = pid_refs[i]
            rvec = jnp.full((16,), i, dtype=jnp.int32)
            for c0 in (0, 16):
                col = jnp.minimum(c0 + lane, NQ - 1) * 4
                t0 = plsc.load_gather(raw_v, [rvec, col])
                t1 = plsc.load_gather(raw_v, [rvec, col + 1])
                t2 = plsc.load_gather(raw_v, [rvec, col + 2])
                t3 = plsc.load_gather(raw_v, [rvec, col + 3])
                pidr[pl.ds(c0, 16)] = ((t0 * 4 + t1) * 4 + t2) * 4 + t3

    def fire_and_drain_in(b, base):
        descs = []
        for i in range(CH):
            descs.append(
                pltpu.async_copy(cont_hbm.at[base + i], buf_v.at[b, i, 0], sem_in)
            )
            descs.append(
                pltpu.async_copy(
                    qtable_hbm.at[pid_refs[i].at[pl.ds(0, NQ)]],
                    buf_v.at[b, i, pl.ds(1, NQ)],
                    sem_in,
                )
            )
        for d in descs:
            d.wait()

    def fire_out(b, blk):
        for i in range(CH):
            pltpu.async_copy(
                buf_v.at[b, i], out_hbm.at[blk, :, :, i], out_sems[b]
            )

    def drain_out(b, blk):
        for i in range(CH):
            pltpu.make_async_copy(
                buf_v.at[b, i], out_hbm.at[blk, :, :, i], out_sems[b]
            ).wait()

    for b in range(2):
        base = row0 + b * CH
        compute_pids(base)
        fire_and_drain_in(b, base)
        fire_out(b, base // CH)

    def round_body(r, carry):
        for b in range(2):
            base = row0 + (2 * r + b) * CH
            blk = base // CH
            compute_pids(base)
            drain_out(b, blk)
            fire_and_drain_in(b, base)
            fire_out(b, blk)
        return carry

    lax.fori_loop(1, NCHUNK // 2, round_body, 0)

    for b in range(2):
        drain_out(b, (row0 + (NCHUNK - 2 + b) * CH) // CH)


def kernel(continuous_fields, tile_type_field, embed_table):
    idx = tile_type_field.astype(jnp.int32)
    cont = continuous_fields.reshape(B, 2, 128)
    q = jnp.arange(256, dtype=jnp.int32)
    qtable = jnp.concatenate(
        [
            embed_table[jnp.clip((q >> 6) & 3, 0, 2) + 1],
            embed_table[jnp.clip((q >> 4) & 3, 0, 2) + 1],
            embed_table[jnp.clip((q >> 2) & 3, 0, 2) + 1],
            embed_table[jnp.clip(q & 3, 0, 2) + 1],
        ],
        axis=1,
    ).reshape(256, 2, 128)
    mesh = plsc.VectorSubcoreMesh(core_axis_name="c", subcore_axis_name="s")
    out5 = pl.kernel(
        _body,
        out_type=jax.ShapeDtypeStruct((B // 8, NSLOT, 2, 8, 128), jnp.float32),
        mesh=mesh,
        compiler_params=pltpu.CompilerParams(
            use_tc_tiling_on_sc=False, needs_layout_passes=False
        ),
        scratch_types=[
            pltpu.VMEM((CH, L), jnp.int32),
            pltpu.VMEM((2, CH, NSLOT, 2, 128), jnp.float32),
            pltpu.SemaphoreType.DMA,
            pltpu.SemaphoreType.DMA,
            pltpu.SemaphoreType.DMA,
        ]
        + [pltpu.VMEM((NQID,), jnp.int32) for _ in range(CH)],
    )(cont, idx, qtable)
    return out5.transpose(0, 3, 1, 2, 4).reshape(B, NSLOT * 256)
